# Initial kernel scaffold; baseline (speedup 1.0000x reference)
import jax
import jax.numpy as jnp
from jax.experimental import pallas as pl


def kernel(c1_idx, c2_idx, c3_idx, table):
    raise NotImplementedError("write your pallas kernel here")


# speedup vs baseline: 1.1758x; 1.1758x over previous
import functools

import jax
import jax.numpy as jnp
from jax import lax
from jax.experimental import pallas as pl
from jax.experimental.pallas import tpu as pltpu
from jax.experimental.pallas import tpu_sc as plsc

VOCAB = 1000000
EMBED_DIM = 32
BATCH = 4096
SEQ_LEN = 50
L_TOT = 3 * SEQ_LEN

NUM_CORES = 2
NUM_SUBCORES = 16
NUM_WORKERS = NUM_CORES * NUM_SUBCORES
B_PER_W = BATCH // NUM_WORKERS


def _body(c1_ref, c2_ref, c3_ref, table_ref, out_ref,
          idx1_v, idx2_v, idx3_v, rows_v, tbuf_v, sem):
    wid = lax.axis_index("s") * NUM_CORES + lax.axis_index("c")
    base = wid * B_PER_W

    pltpu.sync_copy(c1_ref.at[pl.ds(base, B_PER_W)], idx1_v)
    pltpu.sync_copy(c2_ref.at[pl.ds(base, B_PER_W)], idx2_v)
    pltpu.sync_copy(c3_ref.at[pl.ds(base, B_PER_W)], idx3_v)

    iota_lo = lax.iota(jnp.int32, 16)
    iota_hi = iota_lo + 16

    def per_b(bi, _):
        cp1 = pltpu.async_copy(table_ref.at[idx1_v.at[bi]],
                               rows_v.at[pl.ds(0, SEQ_LEN)], sem)
        cp2 = pltpu.async_copy(table_ref.at[idx2_v.at[bi]],
                               rows_v.at[pl.ds(SEQ_LEN, SEQ_LEN)], sem)
        cp3 = pltpu.async_copy(table_ref.at[idx3_v.at[bi]],
                               rows_v.at[pl.ds(2 * SEQ_LEN, SEQ_LEN)], sem)
        cp1.wait()
        cp2.wait()
        cp3.wait()

        def tr(l, _):
            lv = jnp.full((16,), l, dtype=jnp.int32)
            plsc.store_scatter(tbuf_v, [iota_lo, lv], rows_v[l, 0:16])
            plsc.store_scatter(tbuf_v, [iota_hi, lv], rows_v[l, 16:32])
            return 0

        lax.fori_loop(0, L_TOT, tr, 0, unroll=2)
        pltpu.sync_copy(tbuf_v, out_ref.at[base + bi])
        return 0

    lax.fori_loop(0, B_PER_W, per_b, 0)


@jax.jit
def kernel(c1_idx, c2_idx, c3_idx, table):
    mesh = plsc.VectorSubcoreMesh(
        core_axis_name="c", subcore_axis_name="s",
        num_cores=NUM_CORES, num_subcores=NUM_SUBCORES)
    f = pl.kernel(
        _body,
        out_type=jax.ShapeDtypeStruct((BATCH, EMBED_DIM, L_TOT), jnp.float32),
        mesh=mesh,
        scratch_types=[
            pltpu.VMEM((B_PER_W, SEQ_LEN), jnp.int32),
            pltpu.VMEM((B_PER_W, SEQ_LEN), jnp.int32),
            pltpu.VMEM((B_PER_W, SEQ_LEN), jnp.int32),
            pltpu.VMEM((L_TOT, EMBED_DIM), jnp.float32),
            pltpu.VMEM((EMBED_DIM, L_TOT), jnp.float32),
            pltpu.SemaphoreType.DMA,
        ],
        compiler_params=pltpu.CompilerParams(use_tc_tiling_on_sc=False,
                                             needs_layout_passes=False),
    )
    return f(c1_idx.astype(jnp.int32), c2_idx.astype(jnp.int32),
             c3_idx.astype(jnp.int32), table)


# speedup vs baseline: 1.3696x; 1.1648x over previous
import jax
import jax.numpy as jnp
from jax import lax
from jax.experimental import pallas as pl
from jax.experimental.pallas import tpu as pltpu
from jax.experimental.pallas import tpu_sc as plsc

VOCAB = 1000000
EMBED_DIM = 32
BATCH = 4096
SEQ_LEN = 50
L_TOT = 3 * SEQ_LEN

NUM_CORES = 2
NUM_SUBCORES = 16
NUM_WORKERS = NUM_CORES * NUM_SUBCORES
B_PER_W = BATCH // NUM_WORKERS
GRP = 4
NGRP = B_PER_W // GRP


def _body(c1_ref, c2_ref, c3_ref, table_ref, out_ref,
          idx1_v, idx2_v, idx3_v, rows_a, rows_b, tb_a, tb_b,
          gsem_a, gsem_b, osem_a, osem_b):
    wid = lax.axis_index("s") * NUM_CORES + lax.axis_index("c")
    base = wid * B_PER_W

    pltpu.sync_copy(c1_ref.at[pl.ds(base, B_PER_W)], idx1_v)
    pltpu.sync_copy(c2_ref.at[pl.ds(base, B_PER_W)], idx2_v)
    pltpu.sync_copy(c3_ref.at[pl.ds(base, B_PER_W)], idx3_v)

    iota_lo = lax.iota(jnp.int32, 16)
    iota_hi = iota_lo + 16

    def issue_gathers(g, rows, sem):
        for j in range(GRP):
            b = g * GRP + j
            pltpu.async_copy(table_ref.at[idx1_v.at[b]],
                             rows.at[j, pl.ds(0, SEQ_LEN)], sem)
            pltpu.async_copy(table_ref.at[idx2_v.at[b]],
                             rows.at[j, pl.ds(SEQ_LEN, SEQ_LEN)], sem)
            pltpu.async_copy(table_ref.at[idx3_v.at[b]],
                             rows.at[j, pl.ds(2 * SEQ_LEN, SEQ_LEN)], sem)

    def wait_gathers(g, rows, sem):
        for j in range(GRP):
            b = g * GRP + j
            pltpu.make_async_copy(table_ref.at[idx1_v.at[b]],
                                  rows.at[j, pl.ds(0, SEQ_LEN)], sem).wait()
            pltpu.make_async_copy(table_ref.at[idx2_v.at[b]],
                                  rows.at[j, pl.ds(SEQ_LEN, SEQ_LEN)], sem).wait()
            pltpu.make_async_copy(table_ref.at[idx3_v.at[b]],
                                  rows.at[j, pl.ds(2 * SEQ_LEN, SEQ_LEN)], sem).wait()

    def transpose(rows, tb):
        def tr(l, _):
            lv = jnp.full((16,), l, dtype=jnp.int32)
            for j in range(GRP):
                plsc.store_scatter(tb.at[j], [iota_lo, lv], rows[j, l, 0:16])
                plsc.store_scatter(tb.at[j], [iota_hi, lv], rows[j, l, 16:32])
            return 0
        lax.fori_loop(0, L_TOT, tr, 0, unroll=2)

    def issue_out(g, tb, sem):
        pltpu.async_copy(tb, out_ref.at[pl.ds(base + g * GRP, GRP)], sem)

    def wait_out(g, tb, sem):
        pltpu.make_async_copy(tb, out_ref.at[pl.ds(base + g * GRP, GRP)],
                              sem).wait()

    issue_gathers(0, rows_a, gsem_a)

    def step(i, _):
        ga = 2 * i
        gb = 2 * i + 1
        issue_gathers(gb, rows_b, gsem_b)
        wait_gathers(ga, rows_a, gsem_a)

        @pl.when(i >= 1)
        def _():
            wait_out(ga, tb_a, osem_a)
        transpose(rows_a, tb_a)
        issue_out(ga, tb_a, osem_a)
        @pl.when(i <= (NGRP // 2) - 2)
        def _():
            issue_gathers(gb + 1, rows_a, gsem_a)
        wait_gathers(gb, rows_b, gsem_b)

        @pl.when(i >= 1)
        def _():
            wait_out(gb, tb_b, osem_b)
        transpose(rows_b, tb_b)
        issue_out(gb, tb_b, osem_b)
        return 0

    lax.fori_loop(0, NGRP // 2, step, 0)
    wait_out(NGRP - 2, tb_a, osem_a)
    wait_out(NGRP - 1, tb_b, osem_b)


@jax.jit
def kernel(c1_idx, c2_idx, c3_idx, table):
    mesh = plsc.VectorSubcoreMesh(
        core_axis_name="c", subcore_axis_name="s",
        num_cores=NUM_CORES, num_subcores=NUM_SUBCORES)
    f = pl.kernel(
        _body,
        out_type=jax.ShapeDtypeStruct((BATCH, EMBED_DIM, L_TOT), jnp.float32),
        mesh=mesh,
        scratch_types=[
            pltpu.VMEM((B_PER_W, SEQ_LEN), jnp.int32),
            pltpu.VMEM((B_PER_W, SEQ_LEN), jnp.int32),
            pltpu.VMEM((B_PER_W, SEQ_LEN), jnp.int32),
            pltpu.VMEM((GRP, L_TOT, EMBED_DIM), jnp.float32),
            pltpu.VMEM((GRP, L_TOT, EMBED_DIM), jnp.float32),
            pltpu.VMEM((GRP, EMBED_DIM, L_TOT), jnp.float32),
            pltpu.VMEM((GRP, EMBED_DIM, L_TOT), jnp.float32),
            pltpu.SemaphoreType.DMA,
            pltpu.SemaphoreType.DMA,
            pltpu.SemaphoreType.DMA,
            pltpu.SemaphoreType.DMA,
        ],
        compiler_params=pltpu.CompilerParams(use_tc_tiling_on_sc=False,
                                             needs_layout_passes=False),
    )
    return f(c1_idx.astype(jnp.int32), c2_idx.astype(jnp.int32),
             c3_idx.astype(jnp.int32), table)


# speedup vs baseline: 1.4130x; 1.0317x over previous
import jax
import jax.numpy as jnp
from jax import lax
from jax.experimental import pallas as pl
from jax.experimental.pallas import tpu as pltpu
from jax.experimental.pallas import tpu_sc as plsc

VOCAB = 1000000
EMBED_DIM = 32
BATCH = 4096
SEQ_LEN = 50
L_TOT = 3 * SEQ_LEN
BT = BATCH // 128
NJOBS = L_TOT * BT

NUM_CORES = 2
NUM_SUBCORES = 16
NUM_WORKERS = NUM_CORES * NUM_SUBCORES
JOBS_PER_W = NJOBS // NUM_WORKERS
IDX_PER_W = JOBS_PER_W * 128


def _body(idx_ref, table_ref, out_ref,
          idx_v, rows_a, rows_b, tb_a, tb_b,
          gsem_a, gsem_b, osem_a, osem_b):
    wid = lax.axis_index("s") * NUM_CORES + lax.axis_index("c")
    job0 = wid * JOBS_PER_W

    pltpu.sync_copy(idx_ref.at[pl.ds(job0 * 128, IDX_PER_W)], idx_v)

    iota_lo = lax.iota(jnp.int32, 16)
    iota_hi = iota_lo + 16

    def issue_gather(jloc, rows, sem):
        pltpu.async_copy(table_ref.at[idx_v.at[pl.ds(jloc * 128, 128)]],
                         rows, sem)

    def wait_gather(jloc, rows, sem):
        pltpu.make_async_copy(table_ref.at[idx_v.at[pl.ds(jloc * 128, 128)]],
                              rows, sem).wait()

    def transpose(rows, tb):
        def tr(r, _):
            rv = jnp.full((16,), r, dtype=jnp.int32)
            plsc.store_scatter(tb, [iota_lo, rv], rows[r, 0:16])
            plsc.store_scatter(tb, [iota_hi, rv], rows[r, 16:32])
            return 0
        lax.fori_loop(0, 128, tr, 0, unroll=4)

    def issue_out(jloc, tb, sem):
        job = job0 + jloc
        lt = (job // BT) * 128 + (job % BT)
        for a in range(4):
            pltpu.async_copy(tb.at[pl.ds(8 * a, 8)],
                             out_ref.at[pl.ds((lt + 32 * a) * 8, 8)], sem)

    def wait_out(jloc, tb, sem):
        job = job0 + jloc
        lt = (job // BT) * 128 + (job % BT)
        for a in range(4):
            pltpu.make_async_copy(tb.at[pl.ds(8 * a, 8)],
                                  out_ref.at[pl.ds((lt + 32 * a) * 8, 8)],
                                  sem).wait()

    issue_gather(0, rows_a, gsem_a)

    half = JOBS_PER_W // 2

    def step(i, _):
        ja = 2 * i
        jb = 2 * i + 1
        issue_gather(jb, rows_b, gsem_b)
        wait_gather(ja, rows_a, gsem_a)

        @pl.when(i >= 1)
        def _():
            wait_out(ja - 2, tb_a, osem_a)
        transpose(rows_a, tb_a)
        issue_out(ja, tb_a, osem_a)
        @pl.when(i <= half - 2)
        def _():
            issue_gather(jb + 1, rows_a, gsem_a)
        wait_gather(jb, rows_b, gsem_b)

        @pl.when(i >= 1)
        def _():
            wait_out(jb - 2, tb_b, osem_b)
        transpose(rows_b, tb_b)
        issue_out(jb, tb_b, osem_b)
        return 0

    lax.fori_loop(0, half, step, 0)
    wait_out(JOBS_PER_W - 2, tb_a, osem_a)
    wait_out(JOBS_PER_W - 1, tb_b, osem_b)


@jax.jit
def kernel(c1_idx, c2_idx, c3_idx, table):
    idx_flat = jnp.concatenate(
        [c1_idx.T.astype(jnp.int32), c2_idx.T.astype(jnp.int32),
         c3_idx.T.astype(jnp.int32)], axis=0).reshape(NJOBS * 128)

    mesh = plsc.VectorSubcoreMesh(
        core_axis_name="c", subcore_axis_name="s",
        num_cores=NUM_CORES, num_subcores=NUM_SUBCORES)
    out2 = pl.kernel(
        _body,
        out_type=jax.ShapeDtypeStruct((NJOBS * 32, 128), jnp.float32),
        mesh=mesh,
        scratch_types=[
            pltpu.VMEM((IDX_PER_W,), jnp.int32),
            pltpu.VMEM((128, EMBED_DIM), jnp.float32),
            pltpu.VMEM((128, EMBED_DIM), jnp.float32),
            pltpu.VMEM((EMBED_DIM, 128), jnp.float32),
            pltpu.VMEM((EMBED_DIM, 128), jnp.float32),
            pltpu.SemaphoreType.DMA,
            pltpu.SemaphoreType.DMA,
            pltpu.SemaphoreType.DMA,
            pltpu.SemaphoreType.DMA,
        ],
        compiler_params=pltpu.CompilerParams(use_tc_tiling_on_sc=False,
                                             needs_layout_passes=False),
    )(idx_flat, table)

    return (out2.reshape(L_TOT, 4, BT, 8, 128)
            .transpose(2, 4, 1, 3, 0)
            .reshape(BATCH, EMBED_DIM, L_TOT))
